# Initial kernel scaffold; baseline (speedup 1.0000x reference)
import jax
import jax.numpy as jnp
from jax.experimental import pallas as pl


def kernel(x, sim_edge_index, sim_edge_weight, dist_edge_index, dist_edge_weight, W1, b1, W2, b2, Ws, bs, Wd, bd, Wf, bf):
    raise NotImplementedError("write your pallas kernel here")


# speedup vs baseline: 8.1891x; 8.1891x over previous
import jax
import jax.numpy as jnp
from jax import lax
from jax.experimental import pallas as pl
from jax.experimental.pallas import tpu as pltpu
from jax.experimental.pallas import tpu_sc as plsc

_NC = 2
_NS = 16
_L = 16
_B = 80
_ZR = 16

_SC_PARAMS = pltpu.CompilerParams(use_tc_tiling_on_sc=False)
_MESH = dict(core_axis_name="c", subcore_axis_name="s")


def _row_split(n):
    rows_t = (n // _NS) & ~7
    rows_x = n - _NS * rows_t
    assert n % 8 == 0 and rows_t % _ZR == 0 and rows_x % 8 == 0
    return rows_t, rows_x


def _make_deg(n, two_e):
    e_core = two_e // _NC
    e_t = e_core // _NS
    nb = e_t // _B
    rows_t, rows_x = _row_split(n)
    assert e_t % _B == 0

    def body(dst_h, w_h, out_h, dst_v, w_v, zbuf, acc, sem):
        cid = lax.axis_index("c")
        sid = lax.axis_index("s")
        ebase = cid * e_core + sid * e_t

        def zr(j, c):
            zbuf[pl.ds(j * _L, _L)] = jnp.zeros((_L,), jnp.float32)
            return c
        lax.fori_loop(0, rows_t // _L, zr, None)
        pltpu.sync_copy(zbuf, acc.at[pl.ds(sid * rows_t, rows_t)])

        @pl.when(sid == _NS - 1)
        def _():
            pltpu.sync_copy(zbuf.at[pl.ds(0, rows_x)],
                            acc.at[pl.ds(_NS * rows_t, rows_x)])
        plsc.subcore_barrier()

        def block(b, c):
            eoff = ebase + b * _B
            pltpu.sync_copy(dst_h.at[pl.ds(eoff, _B)], dst_v)
            pltpu.sync_copy(w_h.at[pl.ds(eoff, _B)], w_v)
            pltpu.sync_copy(w_v, acc.at[dst_v], add=True)
            return c
        lax.fori_loop(0, nb, block, None)
        plsc.subcore_barrier()

        pltpu.sync_copy(acc.at[pl.ds(sid * rows_t, rows_t)],
                        out_h.at[pl.ds(cid * n + sid * rows_t, rows_t)])

        @pl.when(sid == _NS - 1)
        def _():
            pltpu.sync_copy(acc.at[pl.ds(_NS * rows_t, rows_x)],
                            out_h.at[pl.ds(cid * n + _NS * rows_t, rows_x)])

    return pl.kernel(
        body,
        out_type=jax.ShapeDtypeStruct((_NC * n,), jnp.float32),
        mesh=plsc.VectorSubcoreMesh(**_MESH),
        scratch_types=[
            pltpu.VMEM((_B,), jnp.int32),
            pltpu.VMEM((_B,), jnp.float32),
            pltpu.VMEM((rows_t,), jnp.float32),
            pltpu.VMEM_SHARED((n,), jnp.float32),
            pltpu.SemaphoreType.DMA,
        ],
        compiler_params=_SC_PARAMS,
    )


def _make_edge_scatter(n, d, two_e):
    e_core = two_e // _NC
    e_t = e_core // _NS
    nb = e_t // _B
    rows_t, rows_x = _row_split(n)
    assert e_t % _B == 0
    nchunk = d // _L

    def body(table_h, src_h, dst_h, w_h, out_h,
             src_v, dst_v, w_v, rows_v, zbuf, acc, sem):
        cid = lax.axis_index("c")
        sid = lax.axis_index("s")
        ebase = cid * e_core + sid * e_t

        def zrow(r, c):
            for kk in range(nchunk):
                zbuf[r, pl.ds(kk * _L, _L)] = jnp.zeros((_L,), jnp.float32)
            return c
        lax.fori_loop(0, _ZR, zrow, None)

        def zcp(j, c):
            pltpu.sync_copy(zbuf, acc.at[pl.ds(sid * rows_t + j * _ZR, _ZR)])
            return c
        lax.fori_loop(0, rows_t // _ZR, zcp, None)

        @pl.when(sid == _NS - 1)
        def _():
            def zcpx(j, c):
                pltpu.sync_copy(
                    zbuf.at[pl.ds(0, 8)],
                    acc.at[pl.ds(_NS * rows_t + j * 8, 8)])
                return c
            lax.fori_loop(0, rows_x // 8, zcpx, None)
        plsc.subcore_barrier()

        def block(b, c):
            eoff = ebase + b * _B
            pltpu.sync_copy(dst_h.at[pl.ds(eoff, _B)], dst_v)
            pltpu.sync_copy(w_h.at[pl.ds(eoff, _B)], w_v)
            pltpu.sync_copy(src_h.at[pl.ds(eoff, _B)], src_v)
            pltpu.async_copy(table_h.at[src_v], rows_v, sem).wait()

            def scale_grp(g_, carry):
                wvec = w_v[pl.ds(g_ * _L, _L)]
                for j in range(_L):
                    wv = wvec.at[jnp.full((_L,), j, jnp.int32)].get(
                        mode="promise_in_bounds")
                    e = g_ * _L + j
                    for kk in range(nchunk):
                        sl = pl.ds(kk * _L, _L)
                        rows_v[e, sl] = rows_v[e, sl] * wv
                return carry
            lax.fori_loop(0, _B // _L, scale_grp, None)
            pltpu.sync_copy(rows_v, acc.at[dst_v], add=True)
            return c
        lax.fori_loop(0, nb, block, None)
        plsc.subcore_barrier()

        pltpu.sync_copy(acc.at[pl.ds(sid * rows_t, rows_t)],
                        out_h.at[pl.ds(cid * n + sid * rows_t, rows_t)])

        @pl.when(sid == _NS - 1)
        def _():
            pltpu.sync_copy(acc.at[pl.ds(_NS * rows_t, rows_x)],
                            out_h.at[pl.ds(cid * n + _NS * rows_t, rows_x)])

    return pl.kernel(
        body,
        out_type=jax.ShapeDtypeStruct((_NC * n, d), jnp.float32),
        mesh=plsc.VectorSubcoreMesh(**_MESH),
        scratch_types=[
            pltpu.VMEM((_B,), jnp.int32),
            pltpu.VMEM((_B,), jnp.int32),
            pltpu.VMEM((_B,), jnp.float32),
            pltpu.VMEM((_B, d), jnp.float32),
            pltpu.VMEM((_ZR, d), jnp.float32),
            pltpu.VMEM_SHARED((n, d), jnp.float32),
            pltpu.SemaphoreType.DMA,
        ],
        compiler_params=_SC_PARAMS,
    )


def _tc1_call(x, w1, w2, deg2, n):
    d_hid = w1.shape[1]

    def body(x_ref, w1_ref, w2_ref, deg_ref, g_ref):
        dis = lax.rsqrt(deg_ref[:] + 1.0)
        h1 = jnp.dot(x_ref[:], w1_ref[:], preferred_element_type=jnp.float32)
        h2 = jnp.dot(x_ref[:], w2_ref[:], preferred_element_type=jnp.float32)
        g_ref[:n, :] = dis[:n] * h1
        g_ref[n:, :] = dis[n:] * h2

    return pl.pallas_call(
        body,
        out_shape=jax.ShapeDtypeStruct((_NC * n, d_hid), jnp.float32),
    )(x, w1, w2, deg2)


def _tc2_call(acc1, g1, deg2, b1, b2, ws, wd, n):
    d_out = ws.shape[1]

    def body(acc_ref, g_ref, deg_ref, b1_ref, b2_ref, ws_ref, wd_ref, o_ref):
        dis = lax.rsqrt(deg_ref[:] + 1.0)
        pre = dis * (acc_ref[:] + g_ref[:])
        xs = jnp.maximum(pre[:n] + b1_ref[:], 0.0)
        xd = jnp.maximum(pre[n:] + b2_ref[:], 0.0)
        o_ref[:n, :] = dis[:n] * jnp.dot(
            xs, ws_ref[:], preferred_element_type=jnp.float32)
        o_ref[n:, :] = dis[n:] * jnp.dot(
            xd, wd_ref[:], preferred_element_type=jnp.float32)

    return pl.pallas_call(
        body,
        out_shape=jax.ShapeDtypeStruct((_NC * n, d_out), jnp.float32),
    )(acc1, g1, deg2, b1, b2, ws, wd)


def _tc3_call(acc2, g2, deg2, bs, bd, wf, bf, n):
    d_out = acc2.shape[1]

    def body(acc_ref, g_ref, deg_ref, bs_ref, bd_ref, wf_ref, bf_ref,
             xs_ref, xd_ref, f_ref):
        dis = lax.rsqrt(deg_ref[:] + 1.0)
        t = dis * (acc_ref[:] + g_ref[:])
        x_sim = t[:n] + bs_ref[:]
        x_dist = t[n:] + bd_ref[:]
        xs_ref[:] = x_sim
        xd_ref[:] = x_dist
        f_ref[:] = (jnp.dot(x_sim, wf_ref[:d_out, :],
                            preferred_element_type=jnp.float32)
                    + jnp.dot(x_dist, wf_ref[d_out:, :],
                              preferred_element_type=jnp.float32)
                    + bf_ref[:])

    out_sd = jax.ShapeDtypeStruct((n, d_out), jnp.float32)
    return pl.pallas_call(
        body,
        out_shape=(out_sd, out_sd, out_sd),
    )(acc2, g2, deg2, bs, bd, wf, bf)


def kernel(x, sim_edge_index, sim_edge_weight, dist_edge_index,
           dist_edge_weight, W1, b1, W2, b2, Ws, bs, Wd, bd, Wf, bf):
    n = x.shape[0]
    e = sim_edge_index.shape[1]
    blk = _NS * _B
    e_pad = ((e + blk - 1) // blk) * blk
    pad = e_pad - e

    def prep(ei, ew, off):
        src, dst = ei[0], ei[1]
        if pad:
            src = jnp.concatenate([src, jnp.zeros((pad,), src.dtype)])
            dst = jnp.concatenate([dst, jnp.zeros((pad,), dst.dtype)])
            ew = jnp.concatenate([ew, jnp.zeros((pad,), ew.dtype)])
        return src + off, dst, ew

    s_src, s_dst, s_w = prep(sim_edge_index, sim_edge_weight, 0)
    d_src, d_dst, d_w = prep(dist_edge_index, dist_edge_weight, n)
    src_all = jnp.concatenate([s_src, d_src])
    dst_all = jnp.concatenate([s_dst, d_dst])
    w_all = jnp.concatenate([s_w, d_w])
    two_e = src_all.shape[0]

    deg = _make_deg(n, two_e)(dst_all, w_all)
    deg2 = deg.reshape(_NC * n, 1)
    g1 = _tc1_call(x, W1, W2, deg2, n)
    acc1 = _make_edge_scatter(n, g1.shape[1], two_e)(
        g1, src_all, dst_all, w_all)
    g2 = _tc2_call(acc1, g1, deg2, b1, b2, Ws, Wd, n)
    acc2 = _make_edge_scatter(n, g2.shape[1], two_e)(
        g2, src_all, dst_all, w_all)
    return _tc3_call(acc2, g2, deg2, bs, bd, Wf, bf, n)


# speedup vs baseline: 17.4528x; 2.1312x over previous
import jax
import jax.numpy as jnp
from jax import lax
from jax.experimental import pallas as pl
from jax.experimental.pallas import tpu as pltpu
from jax.experimental.pallas import tpu_sc as plsc

_NC = 2
_NS = 16
_L = 16
_B = 80
_ZR = 16

_SC_PARAMS = pltpu.CompilerParams(use_tc_tiling_on_sc=False)
_MESH = dict(core_axis_name="c", subcore_axis_name="s")


def _row_split(n):
    rows_t = (n // _NS) & ~7
    rows_x = n - _NS * rows_t
    assert n % 8 == 0 and rows_t % _ZR == 0 and rows_x % 8 == 0
    return rows_t, rows_x


def _make_deg(n, two_e):
    e_core = two_e // _NC
    e_t = e_core // _NS
    nb = e_t // _B
    rows_t, rows_x = _row_split(n)
    assert e_t % _B == 0 and nb % 4 == 0

    def body(dst_h, w_h, out_h, *rest):
        dst4 = rest[0:4]
        w4 = rest[4:8]
        zbuf, acc = rest[8:10]
        semi = rest[10:14]
        sems = rest[14:16]

        cid = lax.axis_index("c")
        sid = lax.axis_index("s")
        ebase = cid * e_core + sid * e_t

        def zr(j, c):
            zbuf[pl.ds(j * _L, _L)] = jnp.zeros((_L,), jnp.float32)
            return c
        lax.fori_loop(0, rows_t // _L, zr, None)
        pltpu.sync_copy(zbuf, acc.at[pl.ds(sid * rows_t, rows_t)])

        @pl.when(sid == _NS - 1)
        def _():
            pltpu.sync_copy(zbuf.at[pl.ds(0, rows_x)],
                            acc.at[pl.ds(_NS * rows_t, rows_x)])
        plsc.subcore_barrier()

        def istart(b, p):
            eoff = ebase + b * _B
            pltpu.async_copy(dst_h.at[pl.ds(eoff, _B)], dst4[p], semi[p])
            pltpu.async_copy(w_h.at[pl.ds(eoff, _B)], w4[p], semi[p])

        def iwait(b, p):
            eoff = ebase + b * _B
            pltpu.make_async_copy(dst_h.at[pl.ds(eoff, _B)], dst4[p],
                                  semi[p]).wait()
            pltpu.make_async_copy(w_h.at[pl.ds(eoff, _B)], w4[p],
                                  semi[p]).wait()

        def swait(rp, p):
            pltpu.make_async_copy(w4[p], acc.at[dst4[p]], sems[rp]).wait()

        istart(0, 0)
        istart(1, 1)

        def group(g_, c):
            for p in range(4):
                b = g_ * 4 + p
                rp = p % 2

                @pl.when(b >= 2)
                def _():
                    swait(rp, p)
                iwait(b, p)
                pltpu.async_copy(w4[p], acc.at[dst4[p]], sems[rp], add=True)

                @pl.when(b + 2 < nb)
                def _():
                    istart(b + 2, (p + 2) % 4)
            return c
        lax.fori_loop(0, nb // 4, group, None)
        swait((nb - 2) % 2, (nb - 2) % 4)
        swait((nb - 1) % 2, (nb - 1) % 4)
        plsc.subcore_barrier()

        pltpu.sync_copy(acc.at[pl.ds(sid * rows_t, rows_t)],
                        out_h.at[pl.ds(cid * n + sid * rows_t, rows_t)])

        @pl.when(sid == _NS - 1)
        def _():
            pltpu.sync_copy(acc.at[pl.ds(_NS * rows_t, rows_x)],
                            out_h.at[pl.ds(cid * n + _NS * rows_t, rows_x)])

    return pl.kernel(
        body,
        out_type=jax.ShapeDtypeStruct((_NC * n,), jnp.float32),
        mesh=plsc.VectorSubcoreMesh(**_MESH),
        scratch_types=(
            [pltpu.VMEM((_B,), jnp.int32) for _ in range(4)]
            + [pltpu.VMEM((_B,), jnp.float32) for _ in range(4)]
            + [pltpu.VMEM((rows_t,), jnp.float32),
               pltpu.VMEM_SHARED((n,), jnp.float32)]
            + [pltpu.SemaphoreType.DMA for _ in range(6)]
        ),
        compiler_params=_SC_PARAMS,
    )


def _make_edge_scatter(n, d, two_e):
    e_core = two_e // _NC
    e_t = e_core // _NS
    nb = e_t // _B
    rows_t, rows_x = _row_split(n)
    assert e_t % _B == 0 and nb % 4 == 0
    nchunk = d // _L

    def body(table_h, src_h, dst_h, w_h, out_h, *rest):
        src4 = rest[0:4]
        dst4 = rest[4:8]
        w4 = rest[8:12]
        rows2 = rest[12:14]
        zbuf, acc = rest[14:16]
        semi = rest[16:20]
        semg = rest[20:22]
        sems = rest[22:24]

        cid = lax.axis_index("c")
        sid = lax.axis_index("s")
        ebase = cid * e_core + sid * e_t

        def zrow(r, c):
            for kk in range(nchunk):
                zbuf[r, pl.ds(kk * _L, _L)] = jnp.zeros((_L,), jnp.float32)
            return c
        lax.fori_loop(0, _ZR, zrow, None)

        def zcp(j, c):
            pltpu.sync_copy(zbuf, acc.at[pl.ds(sid * rows_t + j * _ZR, _ZR)])
            return c
        lax.fori_loop(0, rows_t // _ZR, zcp, None)

        @pl.when(sid == _NS - 1)
        def _():
            def zcpx(j, c):
                pltpu.sync_copy(
                    zbuf.at[pl.ds(0, 8)],
                    acc.at[pl.ds(_NS * rows_t + j * 8, 8)])
                return c
            lax.fori_loop(0, rows_x // 8, zcpx, None)
        plsc.subcore_barrier()

        def istart(b, p):
            eoff = ebase + b * _B
            pltpu.async_copy(src_h.at[pl.ds(eoff, _B)], src4[p], semi[p])
            pltpu.async_copy(dst_h.at[pl.ds(eoff, _B)], dst4[p], semi[p])
            pltpu.async_copy(w_h.at[pl.ds(eoff, _B)], w4[p], semi[p])

        def iwait(b, p):
            eoff = ebase + b * _B
            pltpu.make_async_copy(src_h.at[pl.ds(eoff, _B)], src4[p],
                                  semi[p]).wait()
            pltpu.make_async_copy(dst_h.at[pl.ds(eoff, _B)], dst4[p],
                                  semi[p]).wait()
            pltpu.make_async_copy(w_h.at[pl.ds(eoff, _B)], w4[p],
                                  semi[p]).wait()

        def gstart(rp, p):
            pltpu.async_copy(table_h.at[src4[p]], rows2[rp], semg[rp])

        def gwait(rp, p):
            pltpu.make_async_copy(table_h.at[src4[p]], rows2[rp],
                                  semg[rp]).wait()

        def swait(rp, p):
            pltpu.make_async_copy(rows2[rp], acc.at[dst4[p]],
                                  sems[rp]).wait()

        istart(0, 0)
        istart(1, 1)
        istart(2, 2)
        iwait(0, 0)
        gstart(0, 0)

        def group(g_, c):
            for p in range(4):
                b = g_ * 4 + p
                rp = p % 2

                gwait(rp, p)

                @pl.when(b >= 1)
                def _():
                    swait(rp ^ 1, (p - 1) % 4)

                @pl.when(b + 1 < nb)
                def _():
                    iwait(b + 1, (p + 1) % 4)
                    gstart(rp ^ 1, (p + 1) % 4)

                @pl.when(b + 3 < nb)
                def _():
                    istart(b + 3, (p + 3) % 4)

                def scale_grp(gg, carry):
                    wvec = w4[p][pl.ds(gg * _L, _L)]
                    for j in range(_L):
                        wv = wvec.at[jnp.full((_L,), j, jnp.int32)].get(
                            mode="promise_in_bounds")
                        e = gg * _L + j
                        for kk in range(nchunk):
                            sl = pl.ds(kk * _L, _L)
                            rows2[rp][e, sl] = rows2[rp][e, sl] * wv
                    return carry
                lax.fori_loop(0, _B // _L, scale_grp, None)
                pltpu.async_copy(rows2[rp], acc.at[dst4[p]], sems[rp],
                                 add=True)
            return c
        lax.fori_loop(0, nb // 4, group, None)
        swait((nb - 1) % 2, (nb - 1) % 4)
        plsc.subcore_barrier()

        pltpu.sync_copy(acc.at[pl.ds(sid * rows_t, rows_t)],
                        out_h.at[pl.ds(cid * n + sid * rows_t, rows_t)])

        @pl.when(sid == _NS - 1)
        def _():
            pltpu.sync_copy(acc.at[pl.ds(_NS * rows_t, rows_x)],
                            out_h.at[pl.ds(cid * n + _NS * rows_t, rows_x)])

    return pl.kernel(
        body,
        out_type=jax.ShapeDtypeStruct((_NC * n, d), jnp.float32),
        mesh=plsc.VectorSubcoreMesh(**_MESH),
        scratch_types=(
            [pltpu.VMEM((_B,), jnp.int32) for _ in range(4)]
            + [pltpu.VMEM((_B,), jnp.int32) for _ in range(4)]
            + [pltpu.VMEM((_B,), jnp.float32) for _ in range(4)]
            + [pltpu.VMEM((_B, d), jnp.float32) for _ in range(2)]
            + [pltpu.VMEM((_ZR, d), jnp.float32),
               pltpu.VMEM_SHARED((n, d), jnp.float32)]
            + [pltpu.SemaphoreType.DMA for _ in range(8)]
        ),
        compiler_params=_SC_PARAMS,
    )


def _tc1_call(x, w1, w2, deg2, n):
    d_hid = w1.shape[1]

    def body(x_ref, w1_ref, w2_ref, deg_ref, g_ref):
        dis = lax.rsqrt(deg_ref[:] + 1.0)
        h1 = jnp.dot(x_ref[:], w1_ref[:], preferred_element_type=jnp.float32)
        h2 = jnp.dot(x_ref[:], w2_ref[:], preferred_element_type=jnp.float32)
        g_ref[:n, :] = dis[:n] * h1
        g_ref[n:, :] = dis[n:] * h2

    return pl.pallas_call(
        body,
        out_shape=jax.ShapeDtypeStruct((_NC * n, d_hid), jnp.float32),
    )(x, w1, w2, deg2)


def _tc2_call(acc1, g1, deg2, b1, b2, ws, wd, n):
    d_out = ws.shape[1]

    def body(acc_ref, g_ref, deg_ref, b1_ref, b2_ref, ws_ref, wd_ref, o_ref):
        dis = lax.rsqrt(deg_ref[:] + 1.0)
        pre = dis * (acc_ref[:] + g_ref[:])
        xs = jnp.maximum(pre[:n] + b1_ref[:], 0.0)
        xd = jnp.maximum(pre[n:] + b2_ref[:], 0.0)
        o_ref[:n, :] = dis[:n] * jnp.dot(
            xs, ws_ref[:], preferred_element_type=jnp.float32)
        o_ref[n:, :] = dis[n:] * jnp.dot(
            xd, wd_ref[:], preferred_element_type=jnp.float32)

    return pl.pallas_call(
        body,
        out_shape=jax.ShapeDtypeStruct((_NC * n, d_out), jnp.float32),
    )(acc1, g1, deg2, b1, b2, ws, wd)


def _tc3_call(acc2, g2, deg2, bs, bd, wf, bf, n):
    d_out = acc2.shape[1]

    def body(acc_ref, g_ref, deg_ref, bs_ref, bd_ref, wf_ref, bf_ref,
             xs_ref, xd_ref, f_ref):
        dis = lax.rsqrt(deg_ref[:] + 1.0)
        t = dis * (acc_ref[:] + g_ref[:])
        x_sim = t[:n] + bs_ref[:]
        x_dist = t[n:] + bd_ref[:]
        xs_ref[:] = x_sim
        xd_ref[:] = x_dist
        f_ref[:] = (jnp.dot(x_sim, wf_ref[:d_out, :],
                            preferred_element_type=jnp.float32)
                    + jnp.dot(x_dist, wf_ref[d_out:, :],
                              preferred_element_type=jnp.float32)
                    + bf_ref[:])

    out_sd = jax.ShapeDtypeStruct((n, d_out), jnp.float32)
    return pl.pallas_call(
        body,
        out_shape=(out_sd, out_sd, out_sd),
    )(acc2, g2, deg2, bs, bd, wf, bf)


def kernel(x, sim_edge_index, sim_edge_weight, dist_edge_index,
           dist_edge_weight, W1, b1, W2, b2, Ws, bs, Wd, bd, Wf, bf):
    n = x.shape[0]
    e = sim_edge_index.shape[1]
    blk = _NS * _B * 4
    e_pad = ((e + blk - 1) // blk) * blk
    pad = e_pad - e

    def prep(ei, ew, off):
        src, dst = ei[0], ei[1]
        if pad:
            src = jnp.concatenate([src, jnp.zeros((pad,), src.dtype)])
            dst = jnp.concatenate([dst, jnp.zeros((pad,), dst.dtype)])
            ew = jnp.concatenate([ew, jnp.zeros((pad,), ew.dtype)])
        return src + off, dst, ew

    s_src, s_dst, s_w = prep(sim_edge_index, sim_edge_weight, 0)
    d_src, d_dst, d_w = prep(dist_edge_index, dist_edge_weight, n)
    src_all = jnp.concatenate([s_src, d_src])
    dst_all = jnp.concatenate([s_dst, d_dst])
    w_all = jnp.concatenate([s_w, d_w])
    two_e = src_all.shape[0]

    deg = _make_deg(n, two_e)(dst_all, w_all)
    deg2 = deg.reshape(_NC * n, 1)
    g1 = _tc1_call(x, W1, W2, deg2, n)
    acc1 = _make_edge_scatter(n, g1.shape[1], two_e)(
        g1, src_all, dst_all, w_all)
    g2 = _tc2_call(acc1, g1, deg2, b1, b2, Ws, Wd, n)
    acc2 = _make_edge_scatter(n, g2.shape[1], two_e)(
        g2, src_all, dst_all, w_all)
    return _tc3_call(acc2, g2, deg2, bs, bd, Wf, bf, n)
